# baseline (speedup 1.0000x reference)
import functools

import jax
import jax.numpy as jnp
from jax import lax
from jax.experimental import pallas as pl
from jax.experimental.pallas import tpu as pltpu
from jax.experimental.pallas import tpu_sc as plsc


def _dlogit_kernel(h_ref, wt_ref, b_ref, out_ref):
    bd = b_ref[:, 1] - b_ref[:, 0]
    hb = h_ref[...].astype(jnp.bfloat16)
    acc = lax.dot_general(
        hb, wt_ref[...], (((1,), (1,)), ((), ())),
        preferred_element_type=jnp.float32,
    )
    dm = acc + bd[None, :]
    mp2 = dm.shape[1] // 2
    lob = jax.lax.bitcast_convert_type(dm[:, :mp2], jnp.int32)
    hib = jax.lax.bitcast_convert_type(dm[:, mp2:], jnp.int32)
    lor = ((lob + 0x7FFF + ((lob >> 16) & 1)) >> 16) & 0xFFFF
    hir = ((hib + 0x7FFF + ((hib >> 16) & 1)) >> 16) & 0xFFFF
    out_ref[...] = lor | (hir << 16)


def _dlogit_matmul(h, wt, b, Mp):
    N, H = h.shape
    BN = 512
    return pl.pallas_call(
        _dlogit_kernel,
        grid=(N // BN,),
        in_specs=[
            pl.BlockSpec((BN, H), lambda i: (i, 0)),
            pl.BlockSpec((Mp, H), lambda i: (0, 0)),
            pl.BlockSpec((Mp, 2), lambda i: (0, 0)),
        ],
        out_specs=pl.BlockSpec((BN, Mp // 2), lambda i: (i, 0)),
        out_shape=jax.ShapeDtypeStruct((N, Mp // 2), jnp.int32),
    )(h, wt, b)


def _path_loss_sc(dpack, tgt, codes, D):
    N, MP2 = dpack.shape
    V, DP = codes.shape
    info = plsc.get_sparse_core_info()
    NC, NS, L = info.num_cores, info.num_subcores, info.num_lanes
    NW = NC * NS
    TPW = N // NW
    G = TPW // L

    @functools.partial(
        pl.kernel,
        mesh=plsc.VectorSubcoreMesh(core_axis_name="c", subcore_axis_name="s"),
        out_type=jax.ShapeDtypeStruct((NW, L), jnp.float32),
        compiler_params=pltpu.CompilerParams(needs_layout_passes=False),
        scratch_types=[
            pltpu.VMEM((TPW,), jnp.int32),
            pltpu.VMEM((TPW, MP2), jnp.int32),
            pltpu.VMEM((V, DP), jnp.int32),
            pltpu.VMEM((L,), jnp.float32),
            pltpu.SemaphoreType.DMA,
            pltpu.SemaphoreType.DMA,
        ],
    )
    def k(dpack_hbm, tgt_hbm, codes_hbm, out_hbm, tgt_v, d_v, c_v, o_v,
          sem0, sem1):
        wid = lax.axis_index("s") * NC + lax.axis_index("c")
        base = wid * TPW
        HT = TPW // 2
        dcp0 = pltpu.async_copy(
            dpack_hbm.at[pl.ds(base, HT)], d_v.at[pl.ds(0, HT)], sem0)
        dcp1 = pltpu.async_copy(
            dpack_hbm.at[pl.ds(base + HT, HT)], d_v.at[pl.ds(HT, HT)], sem1)
        pltpu.sync_copy(tgt_hbm.at[pl.ds(base, TPW)], tgt_v)
        pltpu.sync_copy(codes_hbm, c_v)

        lanes = lax.iota(jnp.int32, L)
        one = jnp.float32(1.0)
        half = jnp.float32(0.5)
        c3 = jnp.float32(2.99903404e-04)
        c2 = jnp.float32(-5.17901292e-03)
        c1 = jnp.float32(1.24993603e-01)
        c0 = jnp.float32(6.93147497e-01)
        topmask = jnp.full((L,), -65536, jnp.int32)

        def body_g(g, acc):
            tok = g * L + lanes
            v = tgt_v[pl.ds(g * L, L)]
            for j in range(D):
                jv = jnp.full((L,), j, jnp.int32)
                c = plsc.load_gather(c_v, [v, jv])
                maskf = (c & 1).astype(jnp.float32)
                sig = ((c >> 1) & 1).astype(jnp.float32) * 2.0 - one
                m = c >> 2
                pair = plsc.load_gather(d_v, [tok, m & 255])
                dbits = (pair << ((1 - (m >> 8)) * 16)) & topmask
                d = plsc.bitcast(dbits, jnp.float32)
                x = sig * d
                p = one / (one + jnp.exp(-x))
                r = p + p - one
                q = r * r
                s = half + (((c3 * q + c2) * q + c1) * q + c0)
                acc = acc - maskf * (p - s)
            return acc

        dcp0.wait()
        acc = lax.fori_loop(0, G // 2, body_g, jnp.zeros((L,), jnp.float32))
        dcp1.wait()
        acc = lax.fori_loop(G // 2, G, body_g, acc)
        o_v[...] = acc
        pltpu.sync_copy(o_v, out_hbm.at[wid])

    return k(dpack, tgt, codes)


def kernel(hidden, target, W, b, path_nodes, path_bits, path_mask):
    H = hidden.shape[-1]
    h = hidden.reshape(-1, H)
    t = target.reshape(-1).astype(jnp.int32)
    M = W.shape[0]
    Mp = (M + 127) // 128 * 128

    D = path_nodes.shape[1]
    codes = (
        (path_nodes.astype(jnp.int32) << 2)
        | (path_bits.astype(jnp.int32) << 1)
        | path_mask.astype(jnp.int32)
    )
    codes = jnp.pad(codes, ((0, 0), (0, 1)))

    Wr = W.reshape(M, 2 * H)
    wt = (Wr[:, H:] - Wr[:, :H]).astype(jnp.bfloat16)

    dpack = _dlogit_matmul(h, wt, b, Mp)
    partial = _path_loss_sc(dpack, t, codes, D)
    return jnp.sum(partial)


# speedup vs baseline: 1.0106x; 1.0106x over previous
import functools

import jax
import jax.numpy as jnp
from jax import lax
from jax.experimental import pallas as pl
from jax.experimental.pallas import tpu as pltpu
from jax.experimental.pallas import tpu_sc as plsc


def _dlogit_kernel(h_ref, wt_ref, b_ref, out_ref):
    bd = b_ref[:, 1] - b_ref[:, 0]
    hb = h_ref[...].astype(jnp.bfloat16)
    acc = lax.dot_general(
        hb, wt_ref[...], (((1,), (1,)), ((), ())),
        preferred_element_type=jnp.float32,
    )
    dm = acc + bd[None, :]
    mp2 = dm.shape[1] // 2
    lob = jax.lax.bitcast_convert_type(dm[:, :mp2], jnp.int32)
    hib = jax.lax.bitcast_convert_type(dm[:, mp2:], jnp.int32)
    lor = ((lob + 0x7FFF + ((lob >> 16) & 1)) >> 16) & 0xFFFF
    hir = ((hib + 0x7FFF + ((hib >> 16) & 1)) >> 16) & 0xFFFF
    out_ref[...] = lor | (hir << 16)


def _dlogit_matmul(h, wt, b, Mp):
    N, H = h.shape
    BN = 512
    return pl.pallas_call(
        _dlogit_kernel,
        grid=(N // BN,),
        in_specs=[
            pl.BlockSpec((BN, H), lambda i: (i, 0)),
            pl.BlockSpec((Mp, H), lambda i: (0, 0)),
            pl.BlockSpec((Mp, 2), lambda i: (0, 0)),
        ],
        out_specs=pl.BlockSpec((BN, Mp // 2), lambda i: (i, 0)),
        out_shape=jax.ShapeDtypeStruct((N, Mp // 2), jnp.int32),
    )(h, wt, b)


def _path_loss_sc(dpack, tgt, codes, D):
    N, MP2 = dpack.shape
    V, DP = codes.shape
    info = plsc.get_sparse_core_info()
    NC, NS, L = info.num_cores, info.num_subcores, info.num_lanes
    NW = NC * NS
    TPW = N // NW
    G = TPW // L

    @functools.partial(
        pl.kernel,
        mesh=plsc.VectorSubcoreMesh(core_axis_name="c", subcore_axis_name="s"),
        out_type=jax.ShapeDtypeStruct((NW, L), jnp.float32),
        compiler_params=pltpu.CompilerParams(needs_layout_passes=False),
        scratch_types=[
            pltpu.VMEM((TPW,), jnp.int32),
            pltpu.VMEM((TPW, MP2), jnp.int32),
            pltpu.VMEM((V, DP), jnp.int32),
            pltpu.VMEM((L,), jnp.float32),
            pltpu.SemaphoreType.DMA,
        ],
    )
    def k(dpack_hbm, tgt_hbm, codes_hbm, out_hbm, tgt_v, d_v, c_v, o_v, sem):
        wid = lax.axis_index("s") * NC + lax.axis_index("c")
        base = wid * TPW
        dcp = pltpu.async_copy(dpack_hbm.at[pl.ds(base, TPW)], d_v, sem)
        pltpu.sync_copy(tgt_hbm.at[pl.ds(base, TPW)], tgt_v)
        pltpu.sync_copy(codes_hbm, c_v)
        dcp.wait()

        lanes = lax.iota(jnp.int32, L)
        one = jnp.float32(1.0)
        half = jnp.float32(0.5)
        c3 = jnp.float32(2.99903404e-04)
        c2 = jnp.float32(-5.17901292e-03)
        c1 = jnp.float32(1.24993603e-01)
        c0 = jnp.float32(6.93147497e-01)
        topmask = jnp.full((L,), -65536, jnp.int32)

        def body_g(g, acc):
            tok = g * L + lanes
            v = tgt_v[pl.ds(g * L, L)]
            for j in range(D):
                jv = jnp.full((L,), j, jnp.int32)
                c = plsc.load_gather(c_v, [v, jv])
                maskf = (c & 1).astype(jnp.float32)
                sig = ((c >> 1) & 1).astype(jnp.float32) * 2.0 - one
                m = c >> 2
                pair = plsc.load_gather(d_v, [tok, m & 255])
                dbits = (pair << ((1 - (m >> 8)) * 16)) & topmask
                d = plsc.bitcast(dbits, jnp.float32)
                x = sig * d
                p = one / (one + jnp.exp(-x))
                r = p + p - one
                q = r * r
                s = half + (((c3 * q + c2) * q + c1) * q + c0)
                acc = acc - maskf * (p - s)
            return acc

        acc = lax.fori_loop(0, G, body_g, jnp.zeros((L,), jnp.float32))
        o_v[...] = acc
        pltpu.sync_copy(o_v, out_hbm.at[wid])

    return k(dpack, tgt, codes)


def kernel(hidden, target, W, b, path_nodes, path_bits, path_mask):
    H = hidden.shape[-1]
    h = hidden.reshape(-1, H)
    t = target.reshape(-1).astype(jnp.int32)
    M = W.shape[0]
    Mp = (M + 127) // 128 * 128

    D = path_nodes.shape[1]
    codes = (
        (path_nodes.astype(jnp.int32) << 2)
        | (path_bits.astype(jnp.int32) << 1)
        | path_mask.astype(jnp.int32)
    )
    codes = jnp.pad(codes, ((0, 0), (0, 1)))

    Wr = W.reshape(M, 2 * H)
    wt = (Wr[:, H:] - Wr[:, :H]).astype(jnp.bfloat16)

    dpack = _dlogit_matmul(h, wt, b, Mp)
    partial = _path_loss_sc(dpack, t, codes, D)
    return jnp.sum(partial)
